# baseline (speedup 1.0000x reference)
import functools

import jax
import jax.numpy as jnp
from jax import lax
from jax.experimental import pallas as pl
from jax.experimental.pallas import tpu as pltpu
from jax.experimental.pallas import tpu_sc as plsc

N = 10000
E = 320000
D = 128
DO = 16
NC, NS, L = 2, 16, 16
NW = NC * NS
CHUNK = 96
K = 106
E_PAD = NW * K * CHUNK
NPR = 640
N_PAD = NS * NPR

_MESH = plsc.VectorSubcoreMesh(core_axis_name="c", subcore_axis_name="s")
_CP = pltpu.CompilerParams(use_tc_tiling_on_sc=False)


def _seg_sum_sc(g, src3, dst3, zeros2d, d):
    @functools.partial(
        pl.kernel,
        mesh=_MESH,
        out_type=jax.ShapeDtypeStruct((NC, N_PAD, d), jnp.float32),
        compiler_params=_CP,
        scratch_types=[
            pltpu.VMEM((K, CHUNK), jnp.int32),
            pltpu.VMEM((K, CHUNK), jnp.int32),
            pltpu.VMEM((CHUNK, d), jnp.float32),
            pltpu.VMEM((CHUNK, d), jnp.float32),
            pltpu.VMEM_SHARED((N_PAD, d), jnp.float32),
            pltpu.SemaphoreType.DMA,
            pltpu.SemaphoreType.DMA,
        ],
    )
    def k(g_hbm, src_hbm, dst_hbm, z_hbm, out_hbm, idx_s, idx_d, rows0,
          rows1, acc_sh, sem0, sem1):
        cid = lax.axis_index("c")
        sid = lax.axis_index("s")
        wid = sid * NC + cid

        pltpu.sync_copy(src_hbm.at[wid], idx_s)
        pltpu.sync_copy(dst_hbm.at[wid], idx_d)

        pltpu.sync_copy(z_hbm, rows0)
        for t in range(NPR // CHUNK):
            pltpu.sync_copy(
                rows0, acc_sh.at[pl.ds(sid * NPR + t * CHUNK, CHUNK)])
        if NPR % CHUNK:
            pltpu.sync_copy(
                rows0.at[pl.ds(0, NPR % CHUNK)],
                acc_sh.at[pl.ds(sid * NPR + (NPR // CHUNK) * CHUNK,
                                NPR % CHUNK)])
        plsc.subcore_barrier()

        pltpu.async_copy(g_hbm.at[idx_s.at[0]], rows0, sem0)

        @pl.loop(0, K - 2, step=2)
        def _(j):
            pltpu.async_copy(g_hbm.at[idx_s.at[j + 1]], rows1, sem1)
            pltpu.make_async_copy(g_hbm.at[idx_s.at[j]], rows0, sem0).wait()
            pltpu.sync_copy(rows0, acc_sh.at[idx_d.at[j]], add=True)
            pltpu.async_copy(g_hbm.at[idx_s.at[j + 2]], rows0, sem0)
            pltpu.make_async_copy(
                g_hbm.at[idx_s.at[j + 1]], rows1, sem1).wait()
            pltpu.sync_copy(rows1, acc_sh.at[idx_d.at[j + 1]], add=True)

        jl = K - 2
        pltpu.async_copy(g_hbm.at[idx_s.at[jl + 1]], rows1, sem1)
        pltpu.make_async_copy(g_hbm.at[idx_s.at[jl]], rows0, sem0).wait()
        pltpu.sync_copy(rows0, acc_sh.at[idx_d.at[jl]], add=True)
        pltpu.make_async_copy(g_hbm.at[idx_s.at[jl + 1]], rows1, sem1).wait()
        pltpu.sync_copy(rows1, acc_sh.at[idx_d.at[jl + 1]], add=True)

        plsc.subcore_barrier()
        pltpu.sync_copy(acc_sh.at[pl.ds(sid * NPR, NPR)],
                        out_hbm.at[cid, pl.ds(sid * NPR, NPR)])

    return k(g, src3, dst3, zeros2d)


def _deg_sc(dst3, zeros2d, ones2d):
    @functools.partial(
        pl.kernel,
        mesh=_MESH,
        out_type=jax.ShapeDtypeStruct((NC, N_PAD, DO), jnp.float32),
        compiler_params=_CP,
        scratch_types=[
            pltpu.VMEM((K, CHUNK), jnp.int32),
            pltpu.VMEM((CHUNK, DO), jnp.float32),
            pltpu.VMEM_SHARED((N_PAD, DO), jnp.float32),
        ],
    )
    def k(dst_hbm, z_hbm, o_hbm, out_hbm, idx_d, rows_v, acc_sh):
        cid = lax.axis_index("c")
        sid = lax.axis_index("s")
        wid = sid * NC + cid

        pltpu.sync_copy(dst_hbm.at[wid], idx_d)

        pltpu.sync_copy(z_hbm, rows_v)
        for t in range(NPR // CHUNK):
            pltpu.sync_copy(
                rows_v, acc_sh.at[pl.ds(sid * NPR + t * CHUNK, CHUNK)])
        if NPR % CHUNK:
            pltpu.sync_copy(
                rows_v.at[pl.ds(0, NPR % CHUNK)],
                acc_sh.at[pl.ds(sid * NPR + (NPR // CHUNK) * CHUNK,
                                NPR % CHUNK)])
        plsc.subcore_barrier()

        pltpu.sync_copy(o_hbm, rows_v)

        @pl.loop(0, K)
        def _(j):
            pltpu.sync_copy(rows_v, acc_sh.at[idx_d.at[j]], add=True)

        plsc.subcore_barrier()
        pltpu.sync_copy(acc_sh.at[pl.ds(sid * NPR, NPR)],
                        out_hbm.at[cid, pl.ds(sid * NPR, NPR)])

    return k(dst3, zeros2d, ones2d)


_BR = 1000


def _mm_tc(a, w):
    n, din = a.shape
    dout = w.shape[1]

    def body(a_ref, w_ref, o_ref):
        o_ref[...] = lax.dot_general(
            a_ref[...], w_ref[...], (((1,), (0,)), ((), ())),
            preferred_element_type=jnp.float32,
            precision=lax.Precision.HIGHEST)

    return pl.pallas_call(
        body,
        grid=(n // _BR,),
        in_specs=[pl.BlockSpec((_BR, din), lambda i: (i, 0)),
                  pl.BlockSpec((din, dout), lambda i: (0, 0))],
        out_specs=pl.BlockSpec((_BR, dout), lambda i: (i, 0)),
        out_shape=jax.ShapeDtypeStruct((n, dout), jnp.float32),
    )(a, w)


def _dis_g1_tc(degp, h1):
    def body(p_ref, h_ref, g_ref, dis_ref):
        cnt = p_ref[0, :, 0:1] + p_ref[1, :, 0:1]
        dis = lax.rsqrt(cnt + 1.0)
        dis_ref[...] = dis
        g_ref[...] = dis * h_ref[...]

    return pl.pallas_call(
        body,
        grid=(N // _BR,),
        in_specs=[pl.BlockSpec((NC, _BR, DO), lambda i: (0, i, 0)),
                  pl.BlockSpec((_BR, D), lambda i: (i, 0))],
        out_specs=[pl.BlockSpec((_BR, D), lambda i: (i, 0)),
                   pl.BlockSpec((_BR, 1), lambda i: (i, 0))],
        out_shape=[jax.ShapeDtypeStruct((N, D), jnp.float32),
                   jax.ShapeDtypeStruct((N, 1), jnp.float32)],
    )(degp, h1)


def _mid_tc(aggp, g1, dis, b1r, w2p):
    def body(a_ref, g_ref, d_ref, b_ref, w_ref, o_ref):
        agg = a_ref[0] + a_ref[1] + g_ref[...]
        z = jnp.maximum(d_ref[...] * agg + b_ref[...], 0.0)
        o_ref[...] = d_ref[...] * lax.dot_general(
            z, w_ref[...], (((1,), (0,)), ((), ())),
            preferred_element_type=jnp.float32,
            precision=lax.Precision.HIGHEST)

    return pl.pallas_call(
        body,
        grid=(N // _BR,),
        in_specs=[pl.BlockSpec((NC, _BR, D), lambda i: (0, i, 0)),
                  pl.BlockSpec((_BR, D), lambda i: (i, 0)),
                  pl.BlockSpec((_BR, 1), lambda i: (i, 0)),
                  pl.BlockSpec((1, D), lambda i: (0, 0)),
                  pl.BlockSpec((D, DO), lambda i: (0, 0))],
        out_specs=pl.BlockSpec((_BR, DO), lambda i: (i, 0)),
        out_shape=jax.ShapeDtypeStruct((N, DO), jnp.float32),
    )(aggp, g1, dis, b1r, w2p)


def _out_tc(aggp2, g2, dis, b2r):
    def body(a_ref, g_ref, d_ref, b_ref, o_ref):
        o_ref[...] = d_ref[...] * (a_ref[0] + a_ref[1] + g_ref[...]) + b_ref[...]

    return pl.pallas_call(
        body,
        grid=(N // _BR,),
        in_specs=[pl.BlockSpec((NC, _BR, DO), lambda i: (0, i, 0)),
                  pl.BlockSpec((_BR, DO), lambda i: (i, 0)),
                  pl.BlockSpec((_BR, 1), lambda i: (i, 0)),
                  pl.BlockSpec((1, DO), lambda i: (0, 0))],
        out_specs=pl.BlockSpec((_BR, DO), lambda i: (i, 0)),
        out_shape=jax.ShapeDtypeStruct((N, DO), jnp.float32),
    )(aggp2, g2, dis, b2r)


def kernel(x, edge_index, W1, b1, W2, b2):
    src = edge_index[0]
    dst = edge_index[1]
    pad = E_PAD - E
    src3 = jnp.concatenate(
        [src, jnp.zeros((pad,), jnp.int32)]).reshape(NW, K, CHUNK)
    trash = N + (jnp.arange(pad, dtype=jnp.int32) % (N_PAD - N))
    dst3 = jnp.concatenate([dst, trash]).reshape(NW, K, CHUNK)
    zeros_big = jnp.zeros((CHUNK, D), jnp.float32)
    zeros_small = jnp.zeros((CHUNK, DO), jnp.float32)
    ones_small = jnp.ones((CHUNK, DO), jnp.float32)
    w2p = jnp.pad(W2, ((0, 0), (0, DO - W2.shape[1])))
    b1r = b1.reshape(1, D)
    b2r = jnp.pad(b2, (0, DO - b2.shape[0])).reshape(1, DO)

    degp = _deg_sc(dst3, zeros_small, ones_small)
    h1 = _mm_tc(x, W1)
    g1, dis = _dis_g1_tc(degp, h1)
    agg1 = _seg_sum_sc(g1, src3, dst3, zeros_big, D)
    g2 = _mid_tc(agg1, g1, dis, b1r, w2p)
    agg2 = _seg_sum_sc(g2, src3, dst3, zeros_small, DO)
    out16 = _out_tc(agg2, g2, dis, b2r)
    return out16[:, :10]


# speedup vs baseline: 1.1142x; 1.1142x over previous
import functools

import jax
import jax.numpy as jnp
from jax import lax
from jax.experimental import pallas as pl
from jax.experimental.pallas import tpu as pltpu
from jax.experimental.pallas import tpu_sc as plsc

N = 10000
E = 320000
D = 128
DO = 16
NC, NS, L = 2, 16, 16
NW = NC * NS
CHUNK = 128
K = 79
E_PAD = NW * K * CHUNK
NPR = 640
N_PAD = NS * NPR

_MESH = plsc.VectorSubcoreMesh(core_axis_name="c", subcore_axis_name="s")
_CP = pltpu.CompilerParams(use_tc_tiling_on_sc=False)


def _seg_sum_sc(g, src3, dst3, zeros2d, d):
    @functools.partial(
        pl.kernel,
        mesh=_MESH,
        out_type=jax.ShapeDtypeStruct((NC, N_PAD, d), jnp.float32),
        compiler_params=_CP,
        scratch_types=[
            pltpu.VMEM((K, CHUNK), jnp.int32),
            pltpu.VMEM((K, CHUNK), jnp.int32),
            pltpu.VMEM((CHUNK, d), jnp.float32),
            pltpu.VMEM_SHARED((N_PAD, d), jnp.float32),
            pltpu.SemaphoreType.DMA,
        ],
    )
    def k(g_hbm, src_hbm, dst_hbm, z_hbm, out_hbm, idx_s, idx_d, rows0,
          acc_sh, sem0):
        cid = lax.axis_index("c")
        sid = lax.axis_index("s")
        wid = sid * NC + cid

        pltpu.sync_copy(src_hbm.at[wid], idx_s)
        pltpu.sync_copy(dst_hbm.at[wid], idx_d)

        pltpu.sync_copy(z_hbm, rows0)
        for t in range(NPR // CHUNK):
            pltpu.sync_copy(
                rows0, acc_sh.at[pl.ds(sid * NPR + t * CHUNK, CHUNK)])
        if NPR % CHUNK:
            pltpu.sync_copy(
                rows0.at[pl.ds(0, NPR % CHUNK)],
                acc_sh.at[pl.ds(sid * NPR + (NPR // CHUNK) * CHUNK,
                                NPR % CHUNK)])
        plsc.subcore_barrier()

        @pl.loop(0, K)
        def _(j):
            pltpu.async_copy(g_hbm.at[idx_s.at[j]], rows0, sem0).wait()
            pltpu.sync_copy(rows0, acc_sh.at[idx_d.at[j]], add=True)

        plsc.subcore_barrier()
        pltpu.sync_copy(acc_sh.at[pl.ds(sid * NPR, NPR)],
                        out_hbm.at[cid, pl.ds(sid * NPR, NPR)])

    return k(g, src3, dst3, zeros2d)


def _deg_sc(dst3, zeros2d, ones2d):
    @functools.partial(
        pl.kernel,
        mesh=_MESH,
        out_type=jax.ShapeDtypeStruct((NC, N_PAD, DO), jnp.float32),
        compiler_params=_CP,
        scratch_types=[
            pltpu.VMEM((K, CHUNK), jnp.int32),
            pltpu.VMEM((CHUNK, DO), jnp.float32),
            pltpu.VMEM_SHARED((N_PAD, DO), jnp.float32),
        ],
    )
    def k(dst_hbm, z_hbm, o_hbm, out_hbm, idx_d, rows_v, acc_sh):
        cid = lax.axis_index("c")
        sid = lax.axis_index("s")
        wid = sid * NC + cid

        pltpu.sync_copy(dst_hbm.at[wid], idx_d)

        pltpu.sync_copy(z_hbm, rows_v)
        for t in range(NPR // CHUNK):
            pltpu.sync_copy(
                rows_v, acc_sh.at[pl.ds(sid * NPR + t * CHUNK, CHUNK)])
        if NPR % CHUNK:
            pltpu.sync_copy(
                rows_v.at[pl.ds(0, NPR % CHUNK)],
                acc_sh.at[pl.ds(sid * NPR + (NPR // CHUNK) * CHUNK,
                                NPR % CHUNK)])
        plsc.subcore_barrier()

        pltpu.sync_copy(o_hbm, rows_v)

        @pl.loop(0, K)
        def _(j):
            pltpu.sync_copy(rows_v, acc_sh.at[idx_d.at[j]], add=True)

        plsc.subcore_barrier()
        pltpu.sync_copy(acc_sh.at[pl.ds(sid * NPR, NPR)],
                        out_hbm.at[cid, pl.ds(sid * NPR, NPR)])

    return k(dst3, zeros2d, ones2d)


_BR = 1000


def _mm_tc(a, w):
    n, din = a.shape
    dout = w.shape[1]

    def body(a_ref, w_ref, o_ref):
        o_ref[...] = lax.dot_general(
            a_ref[...], w_ref[...], (((1,), (0,)), ((), ())),
            preferred_element_type=jnp.float32,
            precision=lax.Precision.HIGHEST)

    return pl.pallas_call(
        body,
        grid=(n // _BR,),
        in_specs=[pl.BlockSpec((_BR, din), lambda i: (i, 0)),
                  pl.BlockSpec((din, dout), lambda i: (0, 0))],
        out_specs=pl.BlockSpec((_BR, dout), lambda i: (i, 0)),
        out_shape=jax.ShapeDtypeStruct((n, dout), jnp.float32),
    )(a, w)


def _dis_g1_tc(degp, h1):
    def body(p_ref, h_ref, g_ref, dis_ref):
        cnt = p_ref[0, :, 0:1] + p_ref[1, :, 0:1]
        dis = lax.rsqrt(cnt + 1.0)
        dis_ref[...] = dis
        g_ref[...] = dis * h_ref[...]

    return pl.pallas_call(
        body,
        grid=(N // _BR,),
        in_specs=[pl.BlockSpec((NC, _BR, DO), lambda i: (0, i, 0)),
                  pl.BlockSpec((_BR, D), lambda i: (i, 0))],
        out_specs=[pl.BlockSpec((_BR, D), lambda i: (i, 0)),
                   pl.BlockSpec((_BR, 1), lambda i: (i, 0))],
        out_shape=[jax.ShapeDtypeStruct((N, D), jnp.float32),
                   jax.ShapeDtypeStruct((N, 1), jnp.float32)],
    )(degp, h1)


def _mid_tc(aggp, g1, dis, b1r, w2p):
    def body(a_ref, g_ref, d_ref, b_ref, w_ref, o_ref):
        agg = a_ref[0] + a_ref[1] + g_ref[...]
        z = jnp.maximum(d_ref[...] * agg + b_ref[...], 0.0)
        o_ref[...] = d_ref[...] * lax.dot_general(
            z, w_ref[...], (((1,), (0,)), ((), ())),
            preferred_element_type=jnp.float32,
            precision=lax.Precision.HIGHEST)

    return pl.pallas_call(
        body,
        grid=(N // _BR,),
        in_specs=[pl.BlockSpec((NC, _BR, D), lambda i: (0, i, 0)),
                  pl.BlockSpec((_BR, D), lambda i: (i, 0)),
                  pl.BlockSpec((_BR, 1), lambda i: (i, 0)),
                  pl.BlockSpec((1, D), lambda i: (0, 0)),
                  pl.BlockSpec((D, DO), lambda i: (0, 0))],
        out_specs=pl.BlockSpec((_BR, DO), lambda i: (i, 0)),
        out_shape=jax.ShapeDtypeStruct((N, DO), jnp.float32),
    )(aggp, g1, dis, b1r, w2p)


def _out_tc(aggp2, g2, dis, b2r):
    def body(a_ref, g_ref, d_ref, b_ref, o_ref):
        o_ref[...] = d_ref[...] * (a_ref[0] + a_ref[1] + g_ref[...]) + b_ref[...]

    return pl.pallas_call(
        body,
        grid=(N // _BR,),
        in_specs=[pl.BlockSpec((NC, _BR, DO), lambda i: (0, i, 0)),
                  pl.BlockSpec((_BR, DO), lambda i: (i, 0)),
                  pl.BlockSpec((_BR, 1), lambda i: (i, 0)),
                  pl.BlockSpec((1, DO), lambda i: (0, 0))],
        out_specs=pl.BlockSpec((_BR, DO), lambda i: (i, 0)),
        out_shape=jax.ShapeDtypeStruct((N, DO), jnp.float32),
    )(aggp2, g2, dis, b2r)


def kernel(x, edge_index, W1, b1, W2, b2):
    src = edge_index[0]
    dst = edge_index[1]
    pad = E_PAD - E
    src3 = jnp.concatenate(
        [src, jnp.zeros((pad,), jnp.int32)]).reshape(NW, K, CHUNK)
    trash = N + (jnp.arange(pad, dtype=jnp.int32) % (N_PAD - N))
    dst3 = jnp.concatenate([dst, trash]).reshape(NW, K, CHUNK)
    zeros_big = jnp.zeros((CHUNK, D), jnp.float32)
    zeros_small = jnp.zeros((CHUNK, DO), jnp.float32)
    ones_small = jnp.ones((CHUNK, DO), jnp.float32)
    w2p = jnp.pad(W2, ((0, 0), (0, DO - W2.shape[1])))
    b1r = b1.reshape(1, D)
    b2r = jnp.pad(b2, (0, DO - b2.shape[0])).reshape(1, DO)

    degp = _deg_sc(dst3, zeros_small, ones_small)
    h1 = _mm_tc(x, W1)
    g1, dis = _dis_g1_tc(degp, h1)
    agg1 = _seg_sum_sc(g1, src3, dst3, zeros_big, D)
    g2 = _mid_tc(agg1, g1, dis, b1r, w2p)
    agg2 = _seg_sum_sc(g2, src3, dst3, zeros_small, DO)
    out16 = _out_tc(agg2, g2, dis, b2r)
    return out16[:, :10]


# speedup vs baseline: 1.2569x; 1.1282x over previous
import functools

import jax
import jax.numpy as jnp
from jax import lax
from jax.experimental import pallas as pl
from jax.experimental.pallas import tpu as pltpu
from jax.experimental.pallas import tpu_sc as plsc

N = 10000
E = 320000
D = 128
DO = 16
NC, NS, L = 2, 16, 16
NW = NC * NS
CHUNK = 128
KT = 158
TOT = NS * KT
K0_L1, K1_L1 = 102, 56
K0_L2, K1_L2 = 84, 74
K0_DG, K1_DG = 79, 79
T_PAD = NS * K0_L1 + (NS - 1) * K1_L1 + K0_L1
E_PAD = T_PAD * CHUNK
NPR = 640
N_PAD = NS * NPR

_MESH = plsc.VectorSubcoreMesh(core_axis_name="c", subcore_axis_name="s")
_CP = pltpu.CompilerParams(use_tc_tiling_on_sc=False)


def _seg_sum_sc(g, src2, dst2, zeros2d, d, k0, k1):
    @functools.partial(
        pl.kernel,
        mesh=_MESH,
        out_type=jax.ShapeDtypeStruct((NC, N_PAD, d), jnp.float32),
        compiler_params=_CP,
        scratch_types=[
            pltpu.VMEM((k0, CHUNK), jnp.int32),
            pltpu.VMEM((k0, CHUNK), jnp.int32),
            pltpu.VMEM((CHUNK, d), jnp.float32),
            pltpu.VMEM_SHARED((N_PAD, d), jnp.float32),
            pltpu.SemaphoreType.DMA,
        ],
    )
    def k(g_hbm, src_hbm, dst_hbm, z_hbm, out_hbm, idx_s, idx_d, rows0,
          acc_sh, sem0):
        cid = lax.axis_index("c")
        sid = lax.axis_index("s")
        my_k = jnp.where(cid == 0, k0, k1)
        off = jnp.where(cid == 0, sid * k0, NS * k0 + sid * k1)

        pltpu.sync_copy(src_hbm.at[pl.ds(off, k0)], idx_s)
        pltpu.sync_copy(dst_hbm.at[pl.ds(off, k0)], idx_d)

        pltpu.sync_copy(z_hbm, rows0)
        for t in range(NPR // CHUNK):
            pltpu.sync_copy(
                rows0, acc_sh.at[pl.ds(sid * NPR + t * CHUNK, CHUNK)])
        if NPR % CHUNK:
            pltpu.sync_copy(
                rows0.at[pl.ds(0, NPR % CHUNK)],
                acc_sh.at[pl.ds(sid * NPR + (NPR // CHUNK) * CHUNK,
                                NPR % CHUNK)])
        plsc.subcore_barrier()

        @pl.loop(0, my_k)
        def _(j):
            pltpu.async_copy(g_hbm.at[idx_s.at[j]], rows0, sem0).wait()
            pltpu.sync_copy(rows0, acc_sh.at[idx_d.at[j]], add=True)

        plsc.subcore_barrier()
        pltpu.sync_copy(acc_sh.at[pl.ds(sid * NPR, NPR)],
                        out_hbm.at[cid, pl.ds(sid * NPR, NPR)])

    return k(g, src2, dst2, zeros2d)


def _deg_sc(dst2, zeros2d, ones2d):
    @functools.partial(
        pl.kernel,
        mesh=_MESH,
        out_type=jax.ShapeDtypeStruct((NC, N_PAD, DO), jnp.float32),
        compiler_params=_CP,
        scratch_types=[
            pltpu.VMEM((K0_DG, CHUNK), jnp.int32),
            pltpu.VMEM((CHUNK, DO), jnp.float32),
            pltpu.VMEM_SHARED((N_PAD, DO), jnp.float32),
        ],
    )
    def k(dst_hbm, z_hbm, o_hbm, out_hbm, idx_d, rows_v, acc_sh):
        cid = lax.axis_index("c")
        sid = lax.axis_index("s")
        off = jnp.where(cid == 0, sid * K0_DG, NS * K0_DG + sid * K1_DG)

        pltpu.sync_copy(dst_hbm.at[pl.ds(off, K0_DG)], idx_d)

        pltpu.sync_copy(z_hbm, rows_v)
        for t in range(NPR // CHUNK):
            pltpu.sync_copy(
                rows_v, acc_sh.at[pl.ds(sid * NPR + t * CHUNK, CHUNK)])
        if NPR % CHUNK:
            pltpu.sync_copy(
                rows_v.at[pl.ds(0, NPR % CHUNK)],
                acc_sh.at[pl.ds(sid * NPR + (NPR // CHUNK) * CHUNK,
                                NPR % CHUNK)])
        plsc.subcore_barrier()

        pltpu.sync_copy(o_hbm, rows_v)

        @pl.loop(0, K0_DG)
        def _(j):
            pltpu.sync_copy(rows_v, acc_sh.at[idx_d.at[j]], add=True)

        plsc.subcore_barrier()
        pltpu.sync_copy(acc_sh.at[pl.ds(sid * NPR, NPR)],
                        out_hbm.at[cid, pl.ds(sid * NPR, NPR)])

    return k(dst2, zeros2d, ones2d)


_BR = 1000


def _mm_tc(a, w):
    n, din = a.shape
    dout = w.shape[1]

    def body(a_ref, w_ref, o_ref):
        o_ref[...] = lax.dot_general(
            a_ref[...], w_ref[...], (((1,), (0,)), ((), ())),
            preferred_element_type=jnp.float32,
            precision=lax.Precision.HIGHEST)

    return pl.pallas_call(
        body,
        grid=(n // _BR,),
        in_specs=[pl.BlockSpec((_BR, din), lambda i: (i, 0)),
                  pl.BlockSpec((din, dout), lambda i: (0, 0))],
        out_specs=pl.BlockSpec((_BR, dout), lambda i: (i, 0)),
        out_shape=jax.ShapeDtypeStruct((n, dout), jnp.float32),
    )(a, w)


def _dis_g1_tc(degp, h1):
    def body(p_ref, h_ref, g_ref, dis_ref):
        cnt = p_ref[0, :, 0:1] + p_ref[1, :, 0:1]
        dis = lax.rsqrt(cnt + 1.0)
        dis_ref[...] = dis
        g_ref[...] = dis * h_ref[...]

    return pl.pallas_call(
        body,
        grid=(N // _BR,),
        in_specs=[pl.BlockSpec((NC, _BR, DO), lambda i: (0, i, 0)),
                  pl.BlockSpec((_BR, D), lambda i: (i, 0))],
        out_specs=[pl.BlockSpec((_BR, D), lambda i: (i, 0)),
                   pl.BlockSpec((_BR, 1), lambda i: (i, 0))],
        out_shape=[jax.ShapeDtypeStruct((N, D), jnp.float32),
                   jax.ShapeDtypeStruct((N, 1), jnp.float32)],
    )(degp, h1)


def _mid_tc(aggp, g1, dis, b1r, w2p):
    def body(a_ref, g_ref, d_ref, b_ref, w_ref, o_ref):
        agg = a_ref[0] + a_ref[1] + g_ref[...]
        z = jnp.maximum(d_ref[...] * agg + b_ref[...], 0.0)
        o_ref[...] = d_ref[...] * lax.dot_general(
            z, w_ref[...], (((1,), (0,)), ((), ())),
            preferred_element_type=jnp.float32,
            precision=lax.Precision.HIGHEST)

    return pl.pallas_call(
        body,
        grid=(N // _BR,),
        in_specs=[pl.BlockSpec((NC, _BR, D), lambda i: (0, i, 0)),
                  pl.BlockSpec((_BR, D), lambda i: (i, 0)),
                  pl.BlockSpec((_BR, 1), lambda i: (i, 0)),
                  pl.BlockSpec((1, D), lambda i: (0, 0)),
                  pl.BlockSpec((D, DO), lambda i: (0, 0))],
        out_specs=pl.BlockSpec((_BR, DO), lambda i: (i, 0)),
        out_shape=jax.ShapeDtypeStruct((N, DO), jnp.float32),
    )(aggp, g1, dis, b1r, w2p)


def _out_tc(aggp2, g2, dis, b2r):
    def body(a_ref, g_ref, d_ref, b_ref, o_ref):
        o_ref[...] = d_ref[...] * (a_ref[0] + a_ref[1] + g_ref[...]) + b_ref[...]

    return pl.pallas_call(
        body,
        grid=(N // _BR,),
        in_specs=[pl.BlockSpec((NC, _BR, DO), lambda i: (0, i, 0)),
                  pl.BlockSpec((_BR, DO), lambda i: (i, 0)),
                  pl.BlockSpec((_BR, 1), lambda i: (i, 0)),
                  pl.BlockSpec((1, DO), lambda i: (0, 0))],
        out_specs=pl.BlockSpec((_BR, DO), lambda i: (i, 0)),
        out_shape=jax.ShapeDtypeStruct((N, DO), jnp.float32),
    )(aggp2, g2, dis, b2r)


def kernel(x, edge_index, W1, b1, W2, b2):
    src = edge_index[0]
    dst = edge_index[1]
    pad_chunks = T_PAD - E // CHUNK
    src2 = jnp.concatenate(
        [src.reshape(E // CHUNK, CHUNK),
         jnp.zeros((pad_chunks, CHUNK), jnp.int32)])
    trash = jnp.broadcast_to(
        N + jnp.arange(CHUNK, dtype=jnp.int32), (pad_chunks, CHUNK))
    dst2 = jnp.concatenate([dst.reshape(E // CHUNK, CHUNK), trash])
    zeros_big = jnp.zeros((CHUNK, D), jnp.float32)
    zeros_small = jnp.zeros((CHUNK, DO), jnp.float32)
    ones_small = jnp.ones((CHUNK, DO), jnp.float32)
    w2p = jnp.pad(W2, ((0, 0), (0, DO - W2.shape[1])))
    b1r = b1.reshape(1, D)
    b2r = jnp.pad(b2, (0, DO - b2.shape[0])).reshape(1, DO)

    degp = _deg_sc(dst2, zeros_small, ones_small)
    h1 = _mm_tc(x, W1)
    g1, dis = _dis_g1_tc(degp, h1)
    agg1 = _seg_sum_sc(g1, src2, dst2, zeros_big, D, K0_L1, K1_L1)
    g2 = _mid_tc(agg1, g1, dis, b1r, w2p)
    agg2 = _seg_sum_sc(g2, src2, dst2, zeros_small, DO, K0_L2, K1_L2)
    out16 = _out_tc(agg2, g2, dis, b2r)
    return out16[:, :10]


# speedup vs baseline: 1.2810x; 1.0191x over previous
import functools

import jax
import jax.numpy as jnp
from jax import lax
from jax.experimental import pallas as pl
from jax.experimental.pallas import tpu as pltpu
from jax.experimental.pallas import tpu_sc as plsc

N = 10000
E = 320000
D = 128
DO = 16
NC, NS, L = 2, 16, 16
NW = NC * NS
CHUNK = 128
KT = 158
TOT = NS * KT
K0_L1, K1_L1 = 107, 51
K0_L2, K1_L2 = 84, 74
K0_DG, K1_DG = 79, 79
T_PAD = NS * K0_L1 + (NS - 1) * K1_L1 + K0_L1
E_PAD = T_PAD * CHUNK
NPR = 640
N_PAD = NS * NPR

_MESH = plsc.VectorSubcoreMesh(core_axis_name="c", subcore_axis_name="s")
_CP = pltpu.CompilerParams(use_tc_tiling_on_sc=False)


def _seg_sum_sc(g, src2, dst2, zeros2d, d, k0, k1):
    @functools.partial(
        pl.kernel,
        mesh=_MESH,
        out_type=jax.ShapeDtypeStruct((NC, N_PAD, d), jnp.float32),
        compiler_params=_CP,
        scratch_types=[
            pltpu.VMEM((k0, CHUNK), jnp.int32),
            pltpu.VMEM((k0, CHUNK), jnp.int32),
            pltpu.VMEM((CHUNK, d), jnp.float32),
            pltpu.VMEM_SHARED((N_PAD, d), jnp.float32),
            pltpu.SemaphoreType.DMA,
        ],
    )
    def k(g_hbm, src_hbm, dst_hbm, z_hbm, out_hbm, idx_s, idx_d, rows0,
          acc_sh, sem0):
        cid = lax.axis_index("c")
        sid = lax.axis_index("s")
        my_k = jnp.where(cid == 0, k0, k1)
        off = jnp.where(cid == 0, sid * k0, NS * k0 + sid * k1)

        pltpu.sync_copy(src_hbm.at[pl.ds(off, k0)], idx_s)
        pltpu.sync_copy(dst_hbm.at[pl.ds(off, k0)], idx_d)

        pltpu.sync_copy(z_hbm, rows0)
        for t in range(NPR // CHUNK):
            pltpu.sync_copy(
                rows0, acc_sh.at[pl.ds(sid * NPR + t * CHUNK, CHUNK)])
        if NPR % CHUNK:
            pltpu.sync_copy(
                rows0.at[pl.ds(0, NPR % CHUNK)],
                acc_sh.at[pl.ds(sid * NPR + (NPR // CHUNK) * CHUNK,
                                NPR % CHUNK)])
        plsc.subcore_barrier()

        @pl.loop(0, my_k)
        def _(j):
            pltpu.async_copy(g_hbm.at[idx_s.at[j]], rows0, sem0).wait()
            pltpu.sync_copy(rows0, acc_sh.at[idx_d.at[j]], add=True)

        plsc.subcore_barrier()
        pltpu.sync_copy(acc_sh.at[pl.ds(sid * NPR, NPR)],
                        out_hbm.at[cid, pl.ds(sid * NPR, NPR)])

    return k(g, src2, dst2, zeros2d)


def _deg_sc(dst2, zeros2d, ones2d):
    @functools.partial(
        pl.kernel,
        mesh=_MESH,
        out_type=jax.ShapeDtypeStruct((NC, N_PAD, DO), jnp.float32),
        compiler_params=_CP,
        scratch_types=[
            pltpu.VMEM((K0_DG, CHUNK), jnp.int32),
            pltpu.VMEM((CHUNK, DO), jnp.float32),
            pltpu.VMEM_SHARED((N_PAD, DO), jnp.float32),
        ],
    )
    def k(dst_hbm, z_hbm, o_hbm, out_hbm, idx_d, rows_v, acc_sh):
        cid = lax.axis_index("c")
        sid = lax.axis_index("s")
        off = jnp.where(cid == 0, sid * K0_DG, NS * K0_DG + sid * K1_DG)

        pltpu.sync_copy(dst_hbm.at[pl.ds(off, K0_DG)], idx_d)

        pltpu.sync_copy(z_hbm, rows_v)
        for t in range(NPR // CHUNK):
            pltpu.sync_copy(
                rows_v, acc_sh.at[pl.ds(sid * NPR + t * CHUNK, CHUNK)])
        if NPR % CHUNK:
            pltpu.sync_copy(
                rows_v.at[pl.ds(0, NPR % CHUNK)],
                acc_sh.at[pl.ds(sid * NPR + (NPR // CHUNK) * CHUNK,
                                NPR % CHUNK)])
        plsc.subcore_barrier()

        pltpu.sync_copy(o_hbm, rows_v)

        @pl.loop(0, K0_DG)
        def _(j):
            pltpu.sync_copy(rows_v, acc_sh.at[idx_d.at[j]], add=True)

        plsc.subcore_barrier()
        pltpu.sync_copy(acc_sh.at[pl.ds(sid * NPR, NPR)],
                        out_hbm.at[cid, pl.ds(sid * NPR, NPR)])

    return k(dst2, zeros2d, ones2d)


_BR = 1000


def _mm_tc(a, w):
    n, din = a.shape
    dout = w.shape[1]

    def body(a_ref, w_ref, o_ref):
        o_ref[...] = lax.dot_general(
            a_ref[...], w_ref[...], (((1,), (0,)), ((), ())),
            preferred_element_type=jnp.float32,
            precision=lax.Precision.HIGHEST)

    return pl.pallas_call(
        body,
        grid=(n // _BR,),
        in_specs=[pl.BlockSpec((_BR, din), lambda i: (i, 0)),
                  pl.BlockSpec((din, dout), lambda i: (0, 0))],
        out_specs=pl.BlockSpec((_BR, dout), lambda i: (i, 0)),
        out_shape=jax.ShapeDtypeStruct((n, dout), jnp.float32),
    )(a, w)


def _dis_g1_tc(degp, h1):
    def body(p_ref, h_ref, g_ref, dis_ref):
        cnt = p_ref[0, :, 0:1] + p_ref[1, :, 0:1]
        dis = lax.rsqrt(cnt + 1.0)
        dis_ref[...] = dis
        g_ref[...] = dis * h_ref[...]

    return pl.pallas_call(
        body,
        grid=(N // _BR,),
        in_specs=[pl.BlockSpec((NC, _BR, DO), lambda i: (0, i, 0)),
                  pl.BlockSpec((_BR, D), lambda i: (i, 0))],
        out_specs=[pl.BlockSpec((_BR, D), lambda i: (i, 0)),
                   pl.BlockSpec((_BR, 1), lambda i: (i, 0))],
        out_shape=[jax.ShapeDtypeStruct((N, D), jnp.float32),
                   jax.ShapeDtypeStruct((N, 1), jnp.float32)],
    )(degp, h1)


def _mid_tc(aggp, g1, dis, b1r, w2p):
    def body(a_ref, g_ref, d_ref, b_ref, w_ref, o_ref):
        agg = a_ref[0] + a_ref[1] + g_ref[...]
        z = jnp.maximum(d_ref[...] * agg + b_ref[...], 0.0)
        o_ref[...] = d_ref[...] * lax.dot_general(
            z, w_ref[...], (((1,), (0,)), ((), ())),
            preferred_element_type=jnp.float32,
            precision=lax.Precision.HIGHEST)

    return pl.pallas_call(
        body,
        grid=(N // _BR,),
        in_specs=[pl.BlockSpec((NC, _BR, D), lambda i: (0, i, 0)),
                  pl.BlockSpec((_BR, D), lambda i: (i, 0)),
                  pl.BlockSpec((_BR, 1), lambda i: (i, 0)),
                  pl.BlockSpec((1, D), lambda i: (0, 0)),
                  pl.BlockSpec((D, DO), lambda i: (0, 0))],
        out_specs=pl.BlockSpec((_BR, DO), lambda i: (i, 0)),
        out_shape=jax.ShapeDtypeStruct((N, DO), jnp.float32),
    )(aggp, g1, dis, b1r, w2p)


def _out_tc(aggp2, g2, dis, b2r):
    def body(a_ref, g_ref, d_ref, b_ref, o_ref):
        o_ref[...] = d_ref[...] * (a_ref[0] + a_ref[1] + g_ref[...]) + b_ref[...]

    return pl.pallas_call(
        body,
        grid=(N // _BR,),
        in_specs=[pl.BlockSpec((NC, _BR, DO), lambda i: (0, i, 0)),
                  pl.BlockSpec((_BR, DO), lambda i: (i, 0)),
                  pl.BlockSpec((_BR, 1), lambda i: (i, 0)),
                  pl.BlockSpec((1, DO), lambda i: (0, 0))],
        out_specs=pl.BlockSpec((_BR, DO), lambda i: (i, 0)),
        out_shape=jax.ShapeDtypeStruct((N, DO), jnp.float32),
    )(aggp2, g2, dis, b2r)


def kernel(x, edge_index, W1, b1, W2, b2):
    src = edge_index[0]
    dst = edge_index[1]
    pad_chunks = T_PAD - E // CHUNK
    src2 = jnp.concatenate(
        [src.reshape(E // CHUNK, CHUNK),
         jnp.zeros((pad_chunks, CHUNK), jnp.int32)])
    trash = jnp.broadcast_to(
        N + jnp.arange(CHUNK, dtype=jnp.int32), (pad_chunks, CHUNK))
    dst2 = jnp.concatenate([dst.reshape(E // CHUNK, CHUNK), trash])
    zeros_big = jnp.zeros((CHUNK, D), jnp.float32)
    zeros_small = jnp.zeros((CHUNK, DO), jnp.float32)
    ones_small = jnp.ones((CHUNK, DO), jnp.float32)
    w2p = jnp.pad(W2, ((0, 0), (0, DO - W2.shape[1])))
    b1r = b1.reshape(1, D)
    b2r = jnp.pad(b2, (0, DO - b2.shape[0])).reshape(1, DO)

    degp = _deg_sc(dst2, zeros_small, ones_small)
    h1 = _mm_tc(x, W1)
    g1, dis = _dis_g1_tc(degp, h1)
    agg1 = _seg_sum_sc(g1, src2, dst2, zeros_big, D, K0_L1, K1_L1)
    g2 = _mid_tc(agg1, g1, dis, b1r, w2p)
    agg2 = _seg_sum_sc(g2, src2, dst2, zeros_small, DO, K0_L2, K1_L2)
    out16 = _out_tc(agg2, g2, dis, b2r)
    return out16[:, :10]


# speedup vs baseline: 1.4136x; 1.1035x over previous
import functools

import jax
import jax.numpy as jnp
from jax import lax
from jax.experimental import pallas as pl
from jax.experimental.pallas import tpu as pltpu
from jax.experimental.pallas import tpu_sc as plsc

N = 10000
E = 320000
D = 128
DO = 16
NC, NS, L = 2, 16, 16
NW = NC * NS
CHUNK = 128
TOT = E // CHUNK
SPLIT_L1 = (107, 49)
SPLIT_L2 = (83, 73)
SPLIT_DG = (78, 78)
N_BIG = 4
for _k0, _k1a in (SPLIT_L1, SPLIT_L2, SPLIT_DG):
    assert NS * _k0 + NS * _k1a + N_BIG == TOT
NPR = 640
N_PAD = NS * NPR

_MESH = plsc.VectorSubcoreMesh(core_axis_name="c", subcore_axis_name="s")
_CP = pltpu.CompilerParams(use_tc_tiling_on_sc=False)


def _seg_sum_sc(g, src_c, dst_c, zeros2d, d, split):
    k0, k1a = split
    c1 = k1a + 1
    kmax = max(k0, c1)

    @functools.partial(
        pl.kernel,
        mesh=_MESH,
        out_type=jax.ShapeDtypeStruct((NC, N_PAD, d), jnp.float32),
        compiler_params=_CP,
        scratch_types=[
            pltpu.VMEM((kmax, CHUNK), jnp.int32),
            pltpu.VMEM((kmax, CHUNK), jnp.int32),
            pltpu.VMEM((CHUNK, d), jnp.float32),
            pltpu.VMEM_SHARED((N_PAD, d), jnp.float32),
            pltpu.SemaphoreType.DMA,
        ],
    )
    def k(g_hbm, src_hbm, dst_hbm, z_hbm, out_hbm, idx_s, idx_d, rows0,
          acc_sh, sem0):
        cid = lax.axis_index("c")
        sid = lax.axis_index("s")
        my_k = jnp.where(cid == 0, k0,
                         jnp.where(sid < NS - N_BIG, k1a, k1a + 1))
        off = jnp.where(cid == 0, sid * k0,
                        NS * k0 + sid * k1a
                        + jnp.maximum(sid - (NS - N_BIG), 0))

        pltpu.sync_copy(src_hbm.at[pl.ds(off, c1)], idx_s.at[pl.ds(0, c1)])
        pltpu.sync_copy(dst_hbm.at[pl.ds(off, c1)], idx_d.at[pl.ds(0, c1)])
        if k0 > c1:
            @pl.when(cid == 0)
            def _():
                pltpu.sync_copy(src_hbm.at[pl.ds(off + c1, k0 - c1)],
                                idx_s.at[pl.ds(c1, k0 - c1)])
                pltpu.sync_copy(dst_hbm.at[pl.ds(off + c1, k0 - c1)],
                                idx_d.at[pl.ds(c1, k0 - c1)])

        pltpu.sync_copy(z_hbm, rows0)
        for t in range(NPR // CHUNK):
            pltpu.sync_copy(
                rows0, acc_sh.at[pl.ds(sid * NPR + t * CHUNK, CHUNK)])
        if NPR % CHUNK:
            pltpu.sync_copy(
                rows0.at[pl.ds(0, NPR % CHUNK)],
                acc_sh.at[pl.ds(sid * NPR + (NPR // CHUNK) * CHUNK,
                                NPR % CHUNK)])
        plsc.subcore_barrier()

        @pl.loop(0, my_k)
        def _(j):
            pltpu.async_copy(g_hbm.at[idx_s.at[j]], rows0, sem0).wait()
            pltpu.sync_copy(rows0, acc_sh.at[idx_d.at[j]], add=True)

        plsc.subcore_barrier()
        pltpu.sync_copy(acc_sh.at[pl.ds(sid * NPR, NPR)],
                        out_hbm.at[cid, pl.ds(sid * NPR, NPR)])

    return k(g, src_c, dst_c, zeros2d)


def _deg_sc(dst_c, zeros2d, ones2d):
    k0, k1a = SPLIT_DG
    c1 = k1a + 1

    @functools.partial(
        pl.kernel,
        mesh=_MESH,
        out_type=jax.ShapeDtypeStruct((NC, N_PAD, DO), jnp.float32),
        compiler_params=_CP,
        scratch_types=[
            pltpu.VMEM((c1, CHUNK), jnp.int32),
            pltpu.VMEM((CHUNK, DO), jnp.float32),
            pltpu.VMEM_SHARED((N_PAD, DO), jnp.float32),
        ],
    )
    def k(dst_hbm, z_hbm, o_hbm, out_hbm, idx_d, rows_v, acc_sh):
        cid = lax.axis_index("c")
        sid = lax.axis_index("s")
        my_k = jnp.where(cid == 0, k0,
                         jnp.where(sid < NS - N_BIG, k1a, k1a + 1))
        off = jnp.where(cid == 0, sid * k0,
                        NS * k0 + sid * k1a
                        + jnp.maximum(sid - (NS - N_BIG), 0))
        pltpu.sync_copy(dst_hbm.at[pl.ds(off, c1)], idx_d)

        pltpu.sync_copy(z_hbm, rows_v)
        for t in range(NPR // CHUNK):
            pltpu.sync_copy(
                rows_v, acc_sh.at[pl.ds(sid * NPR + t * CHUNK, CHUNK)])
        if NPR % CHUNK:
            pltpu.sync_copy(
                rows_v.at[pl.ds(0, NPR % CHUNK)],
                acc_sh.at[pl.ds(sid * NPR + (NPR // CHUNK) * CHUNK,
                                NPR % CHUNK)])
        plsc.subcore_barrier()

        pltpu.sync_copy(o_hbm, rows_v)

        @pl.loop(0, my_k)
        def _(j):
            pltpu.sync_copy(rows_v, acc_sh.at[idx_d.at[j]], add=True)

        plsc.subcore_barrier()
        pltpu.sync_copy(acc_sh.at[pl.ds(sid * NPR, NPR)],
                        out_hbm.at[cid, pl.ds(sid * NPR, NPR)])

    return k(dst_c, zeros2d, ones2d)


_BR = 1000


def _mm_tc(a, w):
    n, din = a.shape
    dout = w.shape[1]

    def body(a_ref, w_ref, o_ref):
        o_ref[...] = lax.dot_general(
            a_ref[...], w_ref[...], (((1,), (0,)), ((), ())),
            preferred_element_type=jnp.float32,
            precision=lax.Precision.HIGHEST)

    return pl.pallas_call(
        body,
        grid=(n // _BR,),
        in_specs=[pl.BlockSpec((_BR, din), lambda i: (i, 0)),
                  pl.BlockSpec((din, dout), lambda i: (0, 0))],
        out_specs=pl.BlockSpec((_BR, dout), lambda i: (i, 0)),
        out_shape=jax.ShapeDtypeStruct((n, dout), jnp.float32),
    )(a, w)


def _dis_g1_tc(degp, h1):
    def body(p_ref, h_ref, g_ref, dis_ref):
        cnt = p_ref[0, :, 0:1] + p_ref[1, :, 0:1]
        dis = lax.rsqrt(cnt + 1.0)
        dis_ref[...] = dis
        g_ref[...] = dis * h_ref[...]

    return pl.pallas_call(
        body,
        grid=(N // _BR,),
        in_specs=[pl.BlockSpec((NC, _BR, DO), lambda i: (0, i, 0)),
                  pl.BlockSpec((_BR, D), lambda i: (i, 0))],
        out_specs=[pl.BlockSpec((_BR, D), lambda i: (i, 0)),
                   pl.BlockSpec((_BR, 1), lambda i: (i, 0))],
        out_shape=[jax.ShapeDtypeStruct((N, D), jnp.float32),
                   jax.ShapeDtypeStruct((N, 1), jnp.float32)],
    )(degp, h1)


def _mid_tc(aggp, g1, dis, b1r, w2p):
    def body(a_ref, g_ref, d_ref, b_ref, w_ref, o_ref):
        agg = a_ref[0] + a_ref[1] + g_ref[...]
        z = jnp.maximum(d_ref[...] * agg + b_ref[...], 0.0)
        o_ref[...] = d_ref[...] * lax.dot_general(
            z, w_ref[...], (((1,), (0,)), ((), ())),
            preferred_element_type=jnp.float32,
            precision=lax.Precision.HIGHEST)

    return pl.pallas_call(
        body,
        grid=(N // _BR,),
        in_specs=[pl.BlockSpec((NC, _BR, D), lambda i: (0, i, 0)),
                  pl.BlockSpec((_BR, D), lambda i: (i, 0)),
                  pl.BlockSpec((_BR, 1), lambda i: (i, 0)),
                  pl.BlockSpec((1, D), lambda i: (0, 0)),
                  pl.BlockSpec((D, DO), lambda i: (0, 0))],
        out_specs=pl.BlockSpec((_BR, DO), lambda i: (i, 0)),
        out_shape=jax.ShapeDtypeStruct((N, DO), jnp.float32),
    )(aggp, g1, dis, b1r, w2p)


def _out_tc(aggp2, g2, dis, b2r):
    def body(a_ref, g_ref, d_ref, b_ref, o_ref):
        o_ref[...] = d_ref[...] * (a_ref[0] + a_ref[1] + g_ref[...]) + b_ref[...]

    return pl.pallas_call(
        body,
        grid=(N // _BR,),
        in_specs=[pl.BlockSpec((NC, _BR, DO), lambda i: (0, i, 0)),
                  pl.BlockSpec((_BR, DO), lambda i: (i, 0)),
                  pl.BlockSpec((_BR, 1), lambda i: (i, 0)),
                  pl.BlockSpec((1, DO), lambda i: (0, 0))],
        out_specs=pl.BlockSpec((_BR, DO), lambda i: (i, 0)),
        out_shape=jax.ShapeDtypeStruct((N, DO), jnp.float32),
    )(aggp2, g2, dis, b2r)


def kernel(x, edge_index, W1, b1, W2, b2):
    ei3 = edge_index.reshape(2, TOT, CHUNK)
    src_c = ei3[0]
    dst_c = ei3[1]
    zeros_big = jnp.zeros((CHUNK, D), jnp.float32)
    zeros_small = jnp.zeros((CHUNK, DO), jnp.float32)
    ones_small = jnp.ones((CHUNK, DO), jnp.float32)
    w2p = jnp.pad(W2, ((0, 0), (0, DO - W2.shape[1])))
    b1r = b1.reshape(1, D)
    b2r = jnp.pad(b2, (0, DO - b2.shape[0])).reshape(1, DO)

    degp = _deg_sc(dst_c, zeros_small, ones_small)
    h1 = _mm_tc(x, W1)
    g1, dis = _dis_g1_tc(degp, h1)
    agg1 = _seg_sum_sc(g1, src_c, dst_c, zeros_big, D, SPLIT_L1)
    g2 = _mid_tc(agg1, g1, dis, b1r, w2p)
    agg2 = _seg_sum_sc(g2, src_c, dst_c, zeros_small, DO, SPLIT_L2)
    out16 = _out_tc(agg2, g2, dis, b2r)
    return out16[:, :10]


# speedup vs baseline: 1.6526x; 1.1690x over previous
import functools

import jax
import jax.numpy as jnp
from jax import lax
from jax.experimental import pallas as pl
from jax.experimental.pallas import tpu as pltpu
from jax.experimental.pallas import tpu_sc as plsc

N = 10000
E = 320000
D = 128
DO = 16
NC, NS, L = 2, 16, 16
NW = NC * NS
CHUNK = 128
TOT = E // CHUNK
SPLIT_L1 = (78, 78)
SPLIT_L2 = (78, 78)
SPLIT_DG = (78, 78)
N_BIG = 4
for _k0, _k1a in (SPLIT_L1, SPLIT_L2, SPLIT_DG):
    assert NS * _k0 + NS * _k1a + N_BIG == TOT
NPR = 640
N_PAD = NS * NPR

_MESH = plsc.VectorSubcoreMesh(core_axis_name="c", subcore_axis_name="s")
_CP = pltpu.CompilerParams(use_tc_tiling_on_sc=False)


def _seg_sum_sc(g, src_c, dst_c, zeros2d, d, split):
    k0, k1a = split
    c1 = k1a + 1
    kmax = max(k0, c1)

    @functools.partial(
        pl.kernel,
        mesh=_MESH,
        out_type=jax.ShapeDtypeStruct((NC, N_PAD, d), jnp.float32),
        compiler_params=_CP,
        scratch_types=[
            pltpu.VMEM((kmax, CHUNK), jnp.int32),
            pltpu.VMEM((kmax, CHUNK), jnp.int32),
            pltpu.VMEM((CHUNK, d), jnp.float32),
            pltpu.VMEM_SHARED((N_PAD, d), jnp.float32),
            pltpu.SemaphoreType.DMA,
        ],
    )
    def k(g_hbm, src_hbm, dst_hbm, z_hbm, out_hbm, idx_s, idx_d, rows0,
          acc_sh, sem0):
        cid = lax.axis_index("c")
        sid = lax.axis_index("s")
        my_k = jnp.where(cid == 0, k0,
                         jnp.where(sid < NS - N_BIG, k1a, k1a + 1))
        off = jnp.where(cid == 0, sid * k0,
                        NS * k0 + sid * k1a
                        + jnp.maximum(sid - (NS - N_BIG), 0))

        pltpu.sync_copy(src_hbm.at[pl.ds(off, c1)], idx_s.at[pl.ds(0, c1)])
        pltpu.sync_copy(dst_hbm.at[pl.ds(off, c1)], idx_d.at[pl.ds(0, c1)])
        if k0 > c1:
            @pl.when(cid == 0)
            def _():
                pltpu.sync_copy(src_hbm.at[pl.ds(off + c1, k0 - c1)],
                                idx_s.at[pl.ds(c1, k0 - c1)])
                pltpu.sync_copy(dst_hbm.at[pl.ds(off + c1, k0 - c1)],
                                idx_d.at[pl.ds(c1, k0 - c1)])

        pltpu.sync_copy(z_hbm, rows0)
        for t in range(NPR // CHUNK):
            pltpu.sync_copy(
                rows0, acc_sh.at[pl.ds(sid * NPR + t * CHUNK, CHUNK)])
        if NPR % CHUNK:
            pltpu.sync_copy(
                rows0.at[pl.ds(0, NPR % CHUNK)],
                acc_sh.at[pl.ds(sid * NPR + (NPR // CHUNK) * CHUNK,
                                NPR % CHUNK)])
        plsc.subcore_barrier()

        @pl.loop(0, my_k)
        def _(j):
            pltpu.async_copy(g_hbm.at[idx_s.at[j]], rows0, sem0).wait()
            pltpu.sync_copy(rows0, acc_sh.at[idx_d.at[j]], add=True)

        plsc.subcore_barrier()
        pltpu.sync_copy(acc_sh.at[pl.ds(sid * NPR, NPR)],
                        out_hbm.at[cid, pl.ds(sid * NPR, NPR)])

    return k(g, src_c, dst_c, zeros2d)


def _deg_sc(dst_c, zeros2d, ones2d):
    k0, k1a = SPLIT_DG
    c1 = k1a + 1

    @functools.partial(
        pl.kernel,
        mesh=_MESH,
        out_type=jax.ShapeDtypeStruct((NC, N_PAD, DO), jnp.float32),
        compiler_params=_CP,
        scratch_types=[
            pltpu.VMEM((c1, CHUNK), jnp.int32),
            pltpu.VMEM((CHUNK, DO), jnp.float32),
            pltpu.VMEM_SHARED((N_PAD, DO), jnp.float32),
        ],
    )
    def k(dst_hbm, z_hbm, o_hbm, out_hbm, idx_d, rows_v, acc_sh):
        cid = lax.axis_index("c")
        sid = lax.axis_index("s")
        my_k = jnp.where(cid == 0, k0,
                         jnp.where(sid < NS - N_BIG, k1a, k1a + 1))
        off = jnp.where(cid == 0, sid * k0,
                        NS * k0 + sid * k1a
                        + jnp.maximum(sid - (NS - N_BIG), 0))
        pltpu.sync_copy(dst_hbm.at[pl.ds(off, c1)], idx_d)

        pltpu.sync_copy(z_hbm, rows_v)
        for t in range(NPR // CHUNK):
            pltpu.sync_copy(
                rows_v, acc_sh.at[pl.ds(sid * NPR + t * CHUNK, CHUNK)])
        if NPR % CHUNK:
            pltpu.sync_copy(
                rows_v.at[pl.ds(0, NPR % CHUNK)],
                acc_sh.at[pl.ds(sid * NPR + (NPR // CHUNK) * CHUNK,
                                NPR % CHUNK)])
        plsc.subcore_barrier()

        pltpu.sync_copy(o_hbm, rows_v)

        @pl.loop(0, my_k)
        def _(j):
            pltpu.sync_copy(rows_v, acc_sh.at[idx_d.at[j]], add=True)

        plsc.subcore_barrier()
        pltpu.sync_copy(acc_sh.at[pl.ds(sid * NPR, NPR)],
                        out_hbm.at[cid, pl.ds(sid * NPR, NPR)])

    return k(dst_c, zeros2d, ones2d)


_BR = 1000


def _mm_tc(a, w):
    n, din = a.shape
    dout = w.shape[1]

    def body(a_ref, w_ref, o_ref):
        o_ref[...] = lax.dot_general(
            a_ref[...], w_ref[...], (((1,), (0,)), ((), ())),
            preferred_element_type=jnp.float32,
            precision=lax.Precision.HIGHEST)

    return pl.pallas_call(
        body,
        grid=(n // _BR,),
        in_specs=[pl.BlockSpec((_BR, din), lambda i: (i, 0)),
                  pl.BlockSpec((din, dout), lambda i: (0, 0))],
        out_specs=pl.BlockSpec((_BR, dout), lambda i: (i, 0)),
        out_shape=jax.ShapeDtypeStruct((n, dout), jnp.float32),
    )(a, w)


def _dis_g1_tc(degp, h1):
    def body(p_ref, h_ref, g_ref, dis_ref):
        cnt = p_ref[0, :, 0:1] + p_ref[1, :, 0:1]
        dis = lax.rsqrt(cnt + 1.0)
        dis_ref[...] = dis
        g_ref[...] = dis * h_ref[...]

    return pl.pallas_call(
        body,
        grid=(N // _BR,),
        in_specs=[pl.BlockSpec((NC, _BR, DO), lambda i: (0, i, 0)),
                  pl.BlockSpec((_BR, D), lambda i: (i, 0))],
        out_specs=[pl.BlockSpec((_BR, D), lambda i: (i, 0)),
                   pl.BlockSpec((_BR, 1), lambda i: (i, 0))],
        out_shape=[jax.ShapeDtypeStruct((N, D), jnp.float32),
                   jax.ShapeDtypeStruct((N, 1), jnp.float32)],
    )(degp, h1)


def _mid_tc(aggp, g1, dis, b1r, w2p):
    def body(a_ref, g_ref, d_ref, b_ref, w_ref, o_ref):
        agg = a_ref[0] + a_ref[1] + g_ref[...]
        z = jnp.maximum(d_ref[...] * agg + b_ref[...], 0.0)
        o_ref[...] = d_ref[...] * lax.dot_general(
            z, w_ref[...], (((1,), (0,)), ((), ())),
            preferred_element_type=jnp.float32,
            precision=lax.Precision.HIGHEST)

    return pl.pallas_call(
        body,
        grid=(N // _BR,),
        in_specs=[pl.BlockSpec((NC, _BR, D), lambda i: (0, i, 0)),
                  pl.BlockSpec((_BR, D), lambda i: (i, 0)),
                  pl.BlockSpec((_BR, 1), lambda i: (i, 0)),
                  pl.BlockSpec((1, D), lambda i: (0, 0)),
                  pl.BlockSpec((D, DO), lambda i: (0, 0))],
        out_specs=pl.BlockSpec((_BR, DO), lambda i: (i, 0)),
        out_shape=jax.ShapeDtypeStruct((N, DO), jnp.float32),
    )(aggp, g1, dis, b1r, w2p)


def _out_tc(aggp2, g2, dis, b2r):
    def body(a_ref, g_ref, d_ref, b_ref, o_ref):
        o_ref[...] = d_ref[...] * (a_ref[0] + a_ref[1] + g_ref[...]) + b_ref[...]

    return pl.pallas_call(
        body,
        grid=(N // _BR,),
        in_specs=[pl.BlockSpec((NC, _BR, DO), lambda i: (0, i, 0)),
                  pl.BlockSpec((_BR, DO), lambda i: (i, 0)),
                  pl.BlockSpec((_BR, 1), lambda i: (i, 0)),
                  pl.BlockSpec((1, DO), lambda i: (0, 0))],
        out_specs=pl.BlockSpec((_BR, DO), lambda i: (i, 0)),
        out_shape=jax.ShapeDtypeStruct((N, DO), jnp.float32),
    )(aggp2, g2, dis, b2r)


def kernel(x, edge_index, W1, b1, W2, b2):
    ei3 = edge_index.reshape(2, TOT, CHUNK)
    src_c = ei3[0]
    dst_c = ei3[1]
    zeros_big = jnp.zeros((CHUNK, D), jnp.float32)
    zeros_small = jnp.zeros((CHUNK, DO), jnp.float32)
    ones_small = jnp.ones((CHUNK, DO), jnp.float32)
    w2p = jnp.pad(W2, ((0, 0), (0, DO - W2.shape[1])))
    b1r = b1.reshape(1, D)
    b2r = jnp.pad(b2, (0, DO - b2.shape[0])).reshape(1, DO)

    degp = _deg_sc(dst_c, zeros_small, ones_small)
    h1 = _mm_tc(x, W1)
    g1, dis = _dis_g1_tc(degp, h1)
    agg1 = _seg_sum_sc(g1, src_c, dst_c, zeros_big, D, SPLIT_L1)
    g2 = _mid_tc(agg1, g1, dis, b1r, w2p)
    agg2 = _seg_sum_sc(g2, src_c, dst_c, zeros_small, DO, SPLIT_L2)
    out16 = _out_tc(agg2, g2, dis, b2r)
    return out16[:, :10]
